# Initial kernel scaffold; baseline (speedup 1.0000x reference)
import jax
import jax.numpy as jnp
from jax.experimental import pallas as pl


def kernel(x, edge_index, mask, W, b):
    raise NotImplementedError("write your pallas kernel here")


# speedup vs baseline: 12.1468x; 12.1468x over previous
import jax
import jax.numpy as jnp
from jax import lax
from jax.experimental import pallas as pl
from jax.experimental.pallas import tpu as pltpu
from jax.experimental.pallas import tpu_sc as plsc

_N = 10000
_E = 320000
_D = 128
_NPAD = 10240
_RPT = _NPAD // 16
_EPW = _E // 32
_EPT = _E // 16
_CH = 128
_NFULL = _EPT // _CH
_REM = _EPT - _NFULL * _CH

_MESH = dict(core_axis_name="c", subcore_axis_name="s",
             num_cores=2, num_subcores=16)


def _deg_body(col_hbm, degp_hbm, degacc, stage, onesb, idxb):
    c = lax.axis_index("c")
    s = lax.axis_index("s")

    def _z(i, _):
        stage[pl.ds(i * 16, 16)] = jnp.zeros((16,), jnp.float32)
        return 0

    lax.fori_loop(0, _RPT // 16, _z, 0)

    def _o(i, _):
        onesb[pl.ds(i * 16, 16)] = jnp.ones((16,), jnp.float32)
        return 0

    lax.fori_loop(0, 5, _o, 0)
    pltpu.sync_copy(stage, degacc.at[pl.ds(s * _RPT, _RPT)])
    plsc.subcore_barrier()

    base = (c * 16 + s) * _EPW

    def _step(j, _):
        pltpu.sync_copy(col_hbm.at[pl.ds(base + j * 80, 80)], idxb)
        pltpu.sync_copy(onesb, degacc.at[idxb], add=True)
        return 0

    lax.fori_loop(0, _EPW // 80, _step, 0)
    plsc.subcore_barrier()
    pltpu.sync_copy(degacc.at[pl.ds(s * _RPT, _RPT)], stage)
    pltpu.sync_copy(stage, degp_hbm.at[c, pl.ds(s * _RPT, _RPT)])


def _make_deg():
    return pl.kernel(
        _deg_body,
        out_type=jax.ShapeDtypeStruct((2, _NPAD), jnp.float32),
        mesh=plsc.VectorSubcoreMesh(**_MESH),
        scratch_types=[
            pltpu.VMEM_SHARED((_NPAD,), jnp.float32),
            pltpu.VMEM((_RPT,), jnp.float32),
            pltpu.VMEM((80,), jnp.float32),
            pltpu.VMEM((80,), jnp.int32),
        ],
    )


_BLK = 1024


def _prescale_body(x_ref, m_ref, degt_ref, yp_ref, mp_ref, dinv_ref):
    dsum = degt_ref[:, 0:1] + degt_ref[:, 1:2]
    dv = jnp.where(dsum > 0, lax.rsqrt(dsum), 0.0)
    mm = m_ref[...]
    yp_ref[...] = x_ref[...] * mm * dv
    mp_ref[...] = mm * dv
    dinv_ref[...] = dv


def _make_prescale():
    return pl.pallas_call(
        _prescale_body,
        grid=(_NPAD // _BLK,),
        in_specs=[
            pl.BlockSpec((_BLK, _D), lambda i: (i, 0)),
            pl.BlockSpec((_BLK, _D), lambda i: (i, 0)),
            pl.BlockSpec((_BLK, 2), lambda i: (i, 0)),
        ],
        out_specs=[
            pl.BlockSpec((_BLK, _D), lambda i: (i, 0)),
            pl.BlockSpec((_BLK, _D), lambda i: (i, 0)),
            pl.BlockSpec((_BLK, 1), lambda i: (i, 0)),
        ],
        out_shape=[
            jax.ShapeDtypeStruct((_N, _D), jnp.float32),
            jax.ShapeDtypeStruct((_N, _D), jnp.float32),
            jax.ShapeDtypeStruct((_NPAD, 1), jnp.float32),
        ],
    )


def _agg_body(yp_hbm, mp_hbm, dinv_hbm, row_hbm, col_hbm,
              t2_hbm, t3_hbm, t1_hbm,
              acc, t1acc, colb, rowb, datab, valsb,
              colr, rowr, datar, valsr):
    c = lax.axis_index("c")
    s = lax.axis_index("s")

    def _zd(t, _):
        datab[t // 8, pl.ds((t % 8) * 16, 16)] = jnp.zeros((16,), jnp.float32)
        return 0

    lax.fori_loop(0, _CH * (_D // 16), _zd, 0)

    def _zv(i, _):
        valsb[pl.ds(i * 16, 16)] = jnp.zeros((16,), jnp.float32)
        return 0

    lax.fori_loop(0, _CH // 16, _zv, 0)

    row0 = s * _RPT
    for q in range(_RPT // _CH):
        pltpu.sync_copy(datab, acc.at[pl.ds(row0 + q * _CH, _CH)])
        pltpu.sync_copy(valsb, t1acc.at[pl.ds(row0 + q * _CH, _CH)])
    plsc.subcore_barrier()

    ebase = s * _EPT

    def _chunk(off, size, cb, rb, db, vb, table, with_t1):
        pltpu.sync_copy(col_hbm.at[pl.ds(off, size)], cb)
        pltpu.sync_copy(row_hbm.at[pl.ds(off, size)], rb)
        pltpu.sync_copy(table.at[cb], db)
        if with_t1:
            pltpu.sync_copy(dinv_hbm.at[cb], vb)
            pltpu.sync_copy(vb, t1acc.at[rb], add=True)
        pltpu.sync_copy(db, acc.at[rb], add=True)

    @pl.when(c == 0)
    def _core0():
        def _stepa(j, _):
            _chunk(ebase + j * _CH, _CH, colb, rowb, datab, valsb, yp_hbm, True)
            return 0

        lax.fori_loop(0, _NFULL, _stepa, 0)
        _chunk(ebase + _NFULL * _CH, _REM, colr, rowr, datar, valsr, yp_hbm, True)

    @pl.when(c == 1)
    def _core1():
        def _stepb(j, _):
            _chunk(ebase + j * _CH, _CH, colb, rowb, datab, valsb, mp_hbm, False)
            return 0

        lax.fori_loop(0, _NFULL, _stepb, 0)
        _chunk(ebase + _NFULL * _CH, _REM, colr, rowr, datar, valsr, mp_hbm, False)

    plsc.subcore_barrier()
    for q in range(_RPT // _CH):
        r = row0 + q * _CH

        @pl.when(c == 0)
        def _out0():
            pltpu.sync_copy(acc.at[pl.ds(r, _CH)], datab)
            pltpu.sync_copy(datab, t2_hbm.at[pl.ds(r, _CH)])
            pltpu.sync_copy(t1acc.at[pl.ds(r, _CH)], valsb)
            pltpu.sync_copy(valsb, t1_hbm.at[pl.ds(r, _CH)])

        @pl.when(c == 1)
        def _out1():
            pltpu.sync_copy(acc.at[pl.ds(r, _CH)], datab)
            pltpu.sync_copy(datab, t3_hbm.at[pl.ds(r, _CH)])


def _make_agg():
    return pl.kernel(
        _agg_body,
        out_type=(
            jax.ShapeDtypeStruct((_NPAD, _D), jnp.float32),
            jax.ShapeDtypeStruct((_NPAD, _D), jnp.float32),
            jax.ShapeDtypeStruct((_NPAD,), jnp.float32),
        ),
        mesh=plsc.VectorSubcoreMesh(**_MESH),
        scratch_types=[
            pltpu.VMEM_SHARED((_NPAD, _D), jnp.float32),
            pltpu.VMEM_SHARED((_NPAD,), jnp.float32),
            pltpu.VMEM((_CH,), jnp.int32),
            pltpu.VMEM((_CH,), jnp.int32),
            pltpu.VMEM((_CH, _D), jnp.float32),
            pltpu.VMEM((_CH,), jnp.float32),
            pltpu.VMEM((_REM,), jnp.int32),
            pltpu.VMEM((_REM,), jnp.int32),
            pltpu.VMEM((_REM, _D), jnp.float32),
            pltpu.VMEM((_REM,), jnp.float32),
        ],
    )


def _final_body(t2_ref, t3_ref, t1_ref, dinv_ref, w_ref, b_ref, o_ref):
    dv = dinv_ref[...]
    t3 = t3_ref[...]
    safe = jnp.where(t3 != 0, t3, 1.0)
    nz = (t3 != 0) & (dv != 0)
    ratio = jnp.where(nz, dv * t1_ref[...] * t2_ref[...] / safe, 0.0)
    o_ref[...] = lax.dot_general(
        ratio, w_ref[...], (((1,), (1,)), ((), ())),
        preferred_element_type=jnp.float32) + b_ref[...]


def _make_final():
    return pl.pallas_call(
        _final_body,
        grid=(_NPAD // _BLK,),
        in_specs=[
            pl.BlockSpec((_BLK, _D), lambda i: (i, 0)),
            pl.BlockSpec((_BLK, _D), lambda i: (i, 0)),
            pl.BlockSpec((_BLK, 1), lambda i: (i, 0)),
            pl.BlockSpec((_BLK, 1), lambda i: (i, 0)),
            pl.BlockSpec((_D, _D), lambda i: (0, 0)),
            pl.BlockSpec((1, _D), lambda i: (0, 0)),
        ],
        out_specs=pl.BlockSpec((_BLK, _D), lambda i: (i, 0)),
        out_shape=jax.ShapeDtypeStruct((_NPAD, _D), jnp.float32),
    )


def kernel(x, edge_index, mask, W, b):
    row = edge_index[0]
    col = edge_index[1]
    degp = _make_deg()(col)
    yp, mp, dinv2 = _make_prescale()(x, mask, degp.T)
    dinv_flat = dinv2.reshape(_NPAD)[:_N]
    t2, t3, t1 = _make_agg()(yp, mp, dinv_flat, row, col)
    out = _make_final()(t2, t3, t1.reshape(_NPAD, 1), dinv2,
                        W, b.reshape(1, _D))
    return out[:_N]


# speedup vs baseline: 13.2427x; 1.0902x over previous
import jax
import jax.numpy as jnp
from jax import lax
from jax.experimental import pallas as pl
from jax.experimental.pallas import tpu as pltpu
from jax.experimental.pallas import tpu_sc as plsc

_N = 10000
_E = 320000
_D = 128
_NPAD = 10240
_RPT = _NPAD // 16
_CH = 128
_EC = 2560
_PADIDX = _NPAD - 1

_MESH = dict(core_axis_name="c", subcore_axis_name="s",
             num_cores=2, num_subcores=16)


_K1_CNT = _EC // 32


def _deg_body(col2_hbm, degp_hbm, degacc, stage, onesb, idxslab, sems):
    c = lax.axis_index("c")
    s = lax.axis_index("s")
    w = c * 16 + s

    def _z(i, _):
        stage[pl.ds(i * 16, 16)] = jnp.zeros((16,), jnp.float32)
        return 0

    lax.fori_loop(0, _RPT // 16, _z, 0)

    def _o(i, _):
        onesb[pl.ds(i * 16, 16)] = jnp.ones((16,), jnp.float32)
        return 0

    lax.fori_loop(0, _CH // 16, _o, 0)
    pltpu.sync_copy(stage, degacc.at[pl.ds(s * _RPT, _RPT)])
    pltpu.sync_copy(col2_hbm.at[pl.ds(w * _K1_CNT, _K1_CNT)], idxslab)
    plsc.subcore_barrier()

    def _fire(j, _):
        pltpu.async_copy(onesb, degacc.at[idxslab.at[j]], sems, add=True)
        return 0

    def _drain(j, _):
        pltpu.make_async_copy(onesb, degacc.at[idxslab.at[0]], sems).wait()
        return 0

    lax.fori_loop(0, _K1_CNT, _fire, 0)
    lax.fori_loop(0, _K1_CNT, _drain, 0)

    plsc.subcore_barrier()
    pltpu.sync_copy(degacc.at[pl.ds(s * _RPT, _RPT)], stage)
    pltpu.sync_copy(stage, degp_hbm.at[c, pl.ds(s * _RPT, _RPT)])


def _make_deg():
    return pl.kernel(
        _deg_body,
        out_type=jax.ShapeDtypeStruct((2, _NPAD), jnp.float32),
        mesh=plsc.VectorSubcoreMesh(**_MESH),
        scratch_types=[
            pltpu.VMEM_SHARED((_NPAD,), jnp.float32),
            pltpu.VMEM((_RPT,), jnp.float32),
            pltpu.VMEM((_CH,), jnp.float32),
            pltpu.VMEM((_K1_CNT, _CH), jnp.int32),
            pltpu.SemaphoreType.DMA,
        ],
    )


_BLK = 1024


def _prescale_body(x_ref, m_ref, degt_ref, yp_ref, mp_ref, dinv_ref):
    dsum = degt_ref[:, 0:1] + degt_ref[:, 1:2]
    dv = jnp.where(dsum > 0, lax.rsqrt(dsum), 0.0)
    mm = m_ref[...]
    yp_ref[...] = x_ref[...] * mm * dv
    mp_ref[...] = mm * dv
    dinv_ref[...] = dv


def _make_prescale():
    return pl.pallas_call(
        _prescale_body,
        grid=(_NPAD // _BLK,),
        in_specs=[
            pl.BlockSpec((_BLK, _D), lambda i: (i, 0)),
            pl.BlockSpec((_BLK, _D), lambda i: (i, 0)),
            pl.BlockSpec((_BLK, 2), lambda i: (i, 0)),
        ],
        out_specs=[
            pl.BlockSpec((_BLK, _D), lambda i: (i, 0)),
            pl.BlockSpec((_BLK, _D), lambda i: (i, 0)),
            pl.BlockSpec((_BLK, 1), lambda i: (i, 0)),
        ],
        out_shape=[
            jax.ShapeDtypeStruct((_NPAD, _D), jnp.float32),
            jax.ShapeDtypeStruct((_NPAD, _D), jnp.float32),
            jax.ShapeDtypeStruct((_NPAD, 1), jnp.float32),
        ],
    )


_K3_CNT = _EC // 16
_SEG = 40
_NSEG = _K3_CNT // _SEG


def _agg_body(yp_hbm, mp_hbm, dinv_hbm, row2_hbm, col2_hbm,
              t2_hbm, t3_hbm, t1_hbm,
              acc, t1acc, colslab, rowslab, datab, valsb,
              sg0, sg1, ss0, ss1, sv0, sv1, st0, st1):
    c = lax.axis_index("c")
    s = lax.axis_index("s")

    def _zd(t, _):
        datab[0, t // 8, pl.ds((t % 8) * 16, 16)] = jnp.zeros((16,), jnp.float32)
        return 0

    lax.fori_loop(0, _CH * (_D // 16), _zd, 0)

    def _zv(i, _):
        valsb[0, pl.ds(i * 16, 16)] = jnp.zeros((16,), jnp.float32)
        return 0

    lax.fori_loop(0, _CH // 16, _zv, 0)

    row0 = s * _RPT
    for q in range(_RPT // _CH):
        pltpu.sync_copy(datab.at[0], acc.at[pl.ds(row0 + q * _CH, _CH)])
        pltpu.sync_copy(valsb.at[0], t1acc.at[pl.ds(row0 + q * _CH, _CH)])
    plsc.subcore_barrier()

    semg = (sg0, sg1)
    sems = (ss0, ss1)
    semv = (sv0, sv1)
    semt = (st0, st1)

    def _mk_ops(table, with_t1):
        def issue_g(j, p):
            pltpu.async_copy(table.at[colslab.at[j]], datab.at[p], semg[p])
            if with_t1:
                pltpu.async_copy(dinv_hbm.at[colslab.at[j]], valsb.at[p], semv[p])

        def wait_g(j, p):
            pltpu.make_async_copy(table.at[colslab.at[j]], datab.at[p],
                                  semg[p]).wait()
            if with_t1:
                pltpu.make_async_copy(dinv_hbm.at[colslab.at[j]], valsb.at[p],
                                      semv[p]).wait()

        def issue_s(j, p):
            pltpu.async_copy(datab.at[p], acc.at[rowslab.at[j]], sems[p],
                             add=True)
            if with_t1:
                pltpu.async_copy(valsb.at[p], t1acc.at[rowslab.at[j]], semt[p],
                                 add=True)

        def wait_s(j, p):
            pltpu.make_async_copy(datab.at[p], acc.at[rowslab.at[j]],
                                  sems[p]).wait()
            if with_t1:
                pltpu.make_async_copy(valsb.at[p], t1acc.at[rowslab.at[j]],
                                      semt[p]).wait()

        return issue_g, wait_g, issue_s, wait_s

    def _pipeline(table, with_t1):
        issue_g, wait_g, issue_s, wait_s = _mk_ops(table, with_t1)
        for seg in range(_NSEG):
            base = s * _K3_CNT + seg * _SEG
            pltpu.sync_copy(col2_hbm.at[pl.ds(base, _SEG)], colslab)
            pltpu.sync_copy(row2_hbm.at[pl.ds(base, _SEG)], rowslab)
            issue_g(0, 0)
            issue_g(1, 1)

            def _steady(jj, _):
                j0 = 2 * jj + 2
                wait_g(j0 - 2, 0)
                issue_s(j0 - 2, 0)
                wait_s(j0 - 2, 0)
                issue_g(j0, 0)
                wait_g(j0 - 1, 1)
                issue_s(j0 - 1, 1)
                wait_s(j0 - 1, 1)
                issue_g(j0 + 1, 1)
                return 0

            lax.fori_loop(0, (_SEG - 2) // 2, _steady, 0)
            wait_g(_SEG - 2, 0)
            issue_s(_SEG - 2, 0)
            wait_g(_SEG - 1, 1)
            wait_s(_SEG - 2, 0)
            issue_s(_SEG - 1, 1)
            wait_s(_SEG - 1, 1)

    @pl.when(c == 0)
    def _():
        _pipeline(yp_hbm, True)

    @pl.when(c == 1)
    def _():
        _pipeline(mp_hbm, False)

    plsc.subcore_barrier()
    for q in range(_RPT // _CH):
        r = row0 + q * _CH

        @pl.when(c == 0)
        def _out0():
            pltpu.sync_copy(acc.at[pl.ds(r, _CH)], datab.at[0])
            pltpu.sync_copy(datab.at[0], t2_hbm.at[pl.ds(r, _CH)])
            pltpu.sync_copy(t1acc.at[pl.ds(r, _CH)], valsb.at[0])
            pltpu.sync_copy(valsb.at[0], t1_hbm.at[pl.ds(r, _CH)])

        @pl.when(c == 1)
        def _out1():
            pltpu.sync_copy(acc.at[pl.ds(r, _CH)], datab.at[0])
            pltpu.sync_copy(datab.at[0], t3_hbm.at[pl.ds(r, _CH)])


def _make_agg():
    return pl.kernel(
        _agg_body,
        out_type=(
            jax.ShapeDtypeStruct((_NPAD, _D), jnp.float32),
            jax.ShapeDtypeStruct((_NPAD, _D), jnp.float32),
            jax.ShapeDtypeStruct((_NPAD,), jnp.float32),
        ),
        mesh=plsc.VectorSubcoreMesh(**_MESH),
        scratch_types=[
            pltpu.VMEM_SHARED((_NPAD, _D), jnp.float32),
            pltpu.VMEM_SHARED((_NPAD,), jnp.float32),
            pltpu.VMEM((_SEG, _CH), jnp.int32),
            pltpu.VMEM((_SEG, _CH), jnp.int32),
            pltpu.VMEM((2, _CH, _D), jnp.float32),
            pltpu.VMEM((2, _CH), jnp.float32),
            pltpu.SemaphoreType.DMA,
            pltpu.SemaphoreType.DMA,
            pltpu.SemaphoreType.DMA,
            pltpu.SemaphoreType.DMA,
            pltpu.SemaphoreType.DMA,
            pltpu.SemaphoreType.DMA,
            pltpu.SemaphoreType.DMA,
            pltpu.SemaphoreType.DMA,
        ],
    )


def _final_body(t2_ref, t3_ref, t1_ref, dinv_ref, w_ref, b_ref, o_ref):
    dv = dinv_ref[...]
    t3 = t3_ref[...]
    safe = jnp.where(t3 != 0, t3, 1.0)
    nz = (t3 != 0) & (dv != 0)
    ratio = jnp.where(nz, dv * t1_ref[...] * t2_ref[...] / safe, 0.0)
    o_ref[...] = lax.dot_general(
        ratio, w_ref[...], (((1,), (1,)), ((), ())),
        preferred_element_type=jnp.float32) + b_ref[...]


def _make_final():
    return pl.pallas_call(
        _final_body,
        grid=(_NPAD // _BLK,),
        in_specs=[
            pl.BlockSpec((_BLK, _D), lambda i: (i, 0)),
            pl.BlockSpec((_BLK, _D), lambda i: (i, 0)),
            pl.BlockSpec((_BLK, 1), lambda i: (i, 0)),
            pl.BlockSpec((_BLK, 1), lambda i: (i, 0)),
            pl.BlockSpec((_D, _D), lambda i: (0, 0)),
            pl.BlockSpec((1, _D), lambda i: (0, 0)),
        ],
        out_specs=pl.BlockSpec((_BLK, _D), lambda i: (i, 0)),
        out_shape=jax.ShapeDtypeStruct((_NPAD, _D), jnp.float32),
    )


def kernel(x, edge_index, mask, W, b):
    npadrows = _EC - _E // _CH
    row2 = jnp.pad(edge_index[0].reshape(-1, _CH), ((0, npadrows), (0, 0)),
                   constant_values=_PADIDX)
    col2 = jnp.pad(edge_index[1].reshape(-1, _CH), ((0, npadrows), (0, 0)),
                   constant_values=_PADIDX)
    degp = _make_deg()(col2)
    yp, mp, dinv2 = _make_prescale()(x, mask, degp.T)
    dinv_flat = dinv2.reshape(_NPAD)
    t2, t3, t1 = _make_agg()(yp, mp, dinv_flat, row2, col2)
    out = _make_final()(t2, t3, t1.reshape(_NPAD, 1), dinv2,
                        W, b.reshape(1, _D))
    return out[:_N]
